# baseline (speedup 1.0000x reference)
import functools

import jax
import jax.numpy as jnp
from jax import lax
from jax.experimental import pallas as pl
from jax.experimental.pallas import tpu as pltpu
from jax.experimental.pallas import tpu_sc as plsc

LENGTH = 5
EMBED_DIM = 768
POOL = 10
TOP_K = 2
N_LAYERS = 5
N_HEADS = 12
BATCH = 64
SEQ = 197

BB = 8
ROW = LENGTH * EMBED_DIM
N_ROWS = N_LAYERS * 2 * BATCH * TOP_K

_NC = 2
_NS = 16
_NW = _NC * _NS
_RPW = N_ROWS // _NW
_CH = 8
_NCH = _RPW // _CH


def _routing_body(x_ref, key_ref, table_ref, sim_ref, idx_ref, bkn_ref,
                  rowid_ref, rsim_ref):
    del table_ref
    i = pl.program_id(0)
    mean = jnp.sum(x_ref[...], axis=0) / jnp.float32(SEQ)
    msq = jnp.sum(mean * mean, axis=1, keepdims=True)
    xn = mean * lax.rsqrt(jnp.maximum(msq, jnp.float32(1e-12)))
    key = key_ref[...]
    ksq = jnp.sum(key * key, axis=1, keepdims=True)
    kn = key * lax.rsqrt(jnp.maximum(ksq, jnp.float32(1e-12)))

    sim = lax.dot_general(xn, kn, (((1,), (1,)), ((), ())),
                          preferred_element_type=jnp.float32)
    sim_ref[...] = sim

    cols = lax.broadcasted_iota(jnp.int32, sim.shape, 1)
    m1 = jnp.max(sim, axis=1, keepdims=True)
    a1 = jnp.min(jnp.where(sim == m1, cols, POOL), axis=1, keepdims=True)
    sim2 = jnp.where(cols == a1, -jnp.inf, sim)
    m2 = jnp.max(sim2, axis=1, keepdims=True)
    a2 = jnp.min(jnp.where(sim2 == m2, cols, POOL), axis=1, keepdims=True)
    idx = jnp.concatenate([a1, a2], axis=1)
    idx_ref[...] = idx

    oh1 = (cols == a1).astype(jnp.float32)
    oh2 = (cols == a2).astype(jnp.float32)
    bkn_ref[:, 0, :] = lax.dot_general(oh1, kn, (((1,), (0,)), ((), ())),
                                       preferred_element_type=jnp.float32)
    bkn_ref[:, 1, :] = lax.dot_general(oh2, kn, (((1,), (0,)), ((), ())),
                                       preferred_element_type=jnp.float32)

    lidx = lax.broadcasted_iota(jnp.int32, (N_LAYERS, 2, BB, TOP_K), 0)
    didx = lax.broadcasted_iota(jnp.int32, (N_LAYERS, 2, BB, TOP_K), 1)
    rowid_ref[...] = (lidx * 2 + didx) * POOL + idx[None, None, :, :]

    part = (jnp.sum(m1) + jnp.sum(m2)) * jnp.float32(1.0 / BATCH)

    @pl.when(i == 0)
    def _():
        rsim_ref[...] = jnp.zeros_like(rsim_ref)

    rsim_ref[...] += part


_routing_call = pl.pallas_call(
    _routing_body,
    grid=(BATCH // BB,),
    in_specs=[
        pl.BlockSpec((SEQ, BB, EMBED_DIM), lambda i: (0, i, 0)),
        pl.BlockSpec((POOL, EMBED_DIM), lambda i: (0, 0)),
        pl.BlockSpec((N_LAYERS * 2 * POOL, ROW), lambda i: (0, 0)),
    ],
    out_specs=[
        pl.BlockSpec((BB, POOL), lambda i: (i, 0)),
        pl.BlockSpec((BB, TOP_K), lambda i: (i, 0)),
        pl.BlockSpec((BB, TOP_K, EMBED_DIM), lambda i: (i, 0, 0)),
        pl.BlockSpec((N_LAYERS, 2, BB, TOP_K), lambda i: (0, 0, i, 0)),
        pl.BlockSpec((1, 128), lambda i: (0, 0)),
    ],
    out_shape=[
        jax.ShapeDtypeStruct((BATCH, POOL), jnp.float32),
        jax.ShapeDtypeStruct((BATCH, TOP_K), jnp.int32),
        jax.ShapeDtypeStruct((BATCH, TOP_K, EMBED_DIM), jnp.float32),
        jax.ShapeDtypeStruct((N_LAYERS, 2, BATCH, TOP_K), jnp.int32),
        jax.ShapeDtypeStruct((1, 128), jnp.float32),
    ],
)


@functools.cache
def _sc_gather_fn():
    mesh = plsc.VectorSubcoreMesh(core_axis_name="c", subcore_axis_name="s")

    @functools.partial(
        pl.kernel,
        mesh=mesh,
        out_type=jax.ShapeDtypeStruct((N_ROWS, ROW), jnp.float32),
        scratch_types=[
            pltpu.VMEM((_RPW,), jnp.int32),
            pltpu.VMEM((16, ROW), jnp.float32),
            pltpu.VMEM((16, ROW), jnp.float32),
            pltpu.SemaphoreType.DMA,
            pltpu.SemaphoreType.DMA,
            pltpu.SemaphoreType.DMA,
            pltpu.SemaphoreType.DMA,
        ],
    )
    def _sc_gather(table_hbm, rowid_hbm, out_hbm, idx_v, rows_a, rows_b,
                   gs0, gs1, ws0, ws1):
        wid = lax.axis_index("s") * _NC + lax.axis_index("c")
        base = wid * _RPW
        pltpu.sync_copy(rowid_hbm.at[pl.ds(base, _RPW)], idx_v)
        a8 = rows_a.at[pl.ds(0, 8)]

        def gstart(off, n, buf, sem):
            return pltpu.async_copy(
                table_hbm.at[idx_v.at[pl.ds(off, n)]], buf, sem)

        g0 = gstart(0, 16, rows_a, gs0)
        g1 = gstart(16, 16, rows_b, gs1)
        g0.wait()
        w0 = pltpu.async_copy(rows_a, out_hbm.at[pl.ds(base, 16)], ws0)
        g1.wait()
        w1 = pltpu.async_copy(rows_b, out_hbm.at[pl.ds(base + 16, 16)], ws1)
        w0.wait()
        g2 = gstart(32, 8, a8, gs0)
        g2.wait()
        w2 = pltpu.async_copy(a8, out_hbm.at[pl.ds(base + 32, 8)], ws0)
        w1.wait()
        w2.wait()

    return _sc_gather


def kernel(x_embed, prompt, prompt_key):
    x_t = jnp.transpose(x_embed, (1, 0, 2))
    table = prompt.reshape(N_LAYERS * 2 * POOL, ROW)
    sim, idx, bkn, rowid, rsim = _routing_call(x_t, prompt_key, table)
    flat = _sc_gather_fn()(table, rowid.reshape(N_ROWS))
    bp = flat.reshape(N_LAYERS, 2, BATCH, TOP_K, LENGTH, N_HEADS,
                      EMBED_DIM // N_HEADS)
    bp = bp.reshape(N_LAYERS, BATCH, 2, TOP_K * LENGTH, N_HEADS,
                    EMBED_DIM // N_HEADS)
    return (bp, sim, idx, bkn, rsim[0, 0])


# speedup vs baseline: 1.0041x; 1.0041x over previous
import functools

import jax
import jax.numpy as jnp
from jax import lax
from jax.experimental import pallas as pl
from jax.experimental.pallas import tpu as pltpu
from jax.experimental.pallas import tpu_sc as plsc

LENGTH = 5
EMBED_DIM = 768
POOL = 10
TOP_K = 2
N_LAYERS = 5
N_HEADS = 12
BATCH = 64
SEQ = 197

BB = 8
ROW = LENGTH * EMBED_DIM
N_ROWS = N_LAYERS * 2 * BATCH * TOP_K

_NC = 2
_NS = 16
_NW = _NC * _NS
_RPW = N_ROWS // _NW


def _routing_body(x_ref, key_ref, sim_ref, idx_ref, bkn_ref,
                  rowid_ref, rsim_ref):
    i = pl.program_id(0)
    mean = jnp.sum(x_ref[...], axis=0) / jnp.float32(SEQ)
    msq = jnp.sum(mean * mean, axis=1, keepdims=True)
    xn = mean * lax.rsqrt(jnp.maximum(msq, jnp.float32(1e-12)))
    key = key_ref[...]
    ksq = jnp.sum(key * key, axis=1, keepdims=True)
    kn = key * lax.rsqrt(jnp.maximum(ksq, jnp.float32(1e-12)))

    sim = lax.dot_general(xn, kn, (((1,), (1,)), ((), ())),
                          preferred_element_type=jnp.float32)
    sim_ref[...] = sim

    cols = lax.broadcasted_iota(jnp.int32, sim.shape, 1)
    m1 = jnp.max(sim, axis=1, keepdims=True)
    a1 = jnp.min(jnp.where(sim == m1, cols, POOL), axis=1, keepdims=True)
    sim2 = jnp.where(cols == a1, -jnp.inf, sim)
    m2 = jnp.max(sim2, axis=1, keepdims=True)
    a2 = jnp.min(jnp.where(sim2 == m2, cols, POOL), axis=1, keepdims=True)
    idx = jnp.concatenate([a1, a2], axis=1)
    idx_ref[...] = idx

    oh1 = (cols == a1).astype(jnp.float32)
    oh2 = (cols == a2).astype(jnp.float32)
    bkn_ref[:, 0, :] = lax.dot_general(oh1, kn, (((1,), (0,)), ((), ())),
                                       preferred_element_type=jnp.float32)
    bkn_ref[:, 1, :] = lax.dot_general(oh2, kn, (((1,), (0,)), ((), ())),
                                       preferred_element_type=jnp.float32)

    lidx = lax.broadcasted_iota(jnp.int32, (N_LAYERS, 2, BB, TOP_K), 0)
    didx = lax.broadcasted_iota(jnp.int32, (N_LAYERS, 2, BB, TOP_K), 1)
    rowid_ref[...] = (lidx * 2 + didx) * POOL + idx[None, None, :, :]

    part = (jnp.sum(m1) + jnp.sum(m2)) * jnp.float32(1.0 / BATCH)

    @pl.when(i == 0)
    def _():
        rsim_ref[...] = jnp.zeros_like(rsim_ref)

    rsim_ref[...] += part


_routing_call = pl.pallas_call(
    _routing_body,
    grid=(BATCH // BB,),
    in_specs=[
        pl.BlockSpec((SEQ, BB, EMBED_DIM), lambda i: (0, i, 0)),
        pl.BlockSpec((POOL, EMBED_DIM), lambda i: (0, 0)),
    ],
    out_specs=[
        pl.BlockSpec((BB, POOL), lambda i: (i, 0)),
        pl.BlockSpec((BB, TOP_K), lambda i: (i, 0)),
        pl.BlockSpec((BB, TOP_K, EMBED_DIM), lambda i: (i, 0, 0)),
        pl.BlockSpec((N_LAYERS, 2, BB, TOP_K), lambda i: (0, 0, i, 0)),
        pl.BlockSpec((1, 128), lambda i: (0, 0)),
    ],
    out_shape=[
        jax.ShapeDtypeStruct((BATCH, POOL), jnp.float32),
        jax.ShapeDtypeStruct((BATCH, TOP_K), jnp.int32),
        jax.ShapeDtypeStruct((BATCH, TOP_K, EMBED_DIM), jnp.float32),
        jax.ShapeDtypeStruct((N_LAYERS, 2, BATCH, TOP_K), jnp.int32),
        jax.ShapeDtypeStruct((1, 128), jnp.float32),
    ],
)


@functools.cache
def _sc_gather_fn():
    mesh = plsc.VectorSubcoreMesh(core_axis_name="c", subcore_axis_name="s")

    @functools.partial(
        pl.kernel,
        mesh=mesh,
        out_type=jax.ShapeDtypeStruct((N_ROWS, ROW), jnp.float32),
        scratch_types=[
            pltpu.VMEM((_RPW,), jnp.int32),
            pltpu.VMEM((16, ROW), jnp.float32),
            pltpu.VMEM((16, ROW), jnp.float32),
            pltpu.SemaphoreType.DMA,
            pltpu.SemaphoreType.DMA,
            pltpu.SemaphoreType.DMA,
            pltpu.SemaphoreType.DMA,
        ],
    )
    def _sc_gather(table_hbm, rowid_hbm, out_hbm, idx_v, rows_a, rows_b,
                   gs0, gs1, ws0, ws1):
        wid = lax.axis_index("s") * _NC + lax.axis_index("c")
        base = wid * _RPW
        pltpu.sync_copy(rowid_hbm.at[pl.ds(base, _RPW)], idx_v)
        a8 = rows_a.at[pl.ds(0, 8)]

        def gstart(off, n, buf, sem):
            return pltpu.async_copy(
                table_hbm.at[idx_v.at[pl.ds(off, n)]], buf, sem)

        g0 = gstart(0, 16, rows_a, gs0)
        g1 = gstart(16, 16, rows_b, gs1)
        g0.wait()
        w0 = pltpu.async_copy(rows_a, out_hbm.at[pl.ds(base, 16)], ws0)
        g1.wait()
        w1 = pltpu.async_copy(rows_b, out_hbm.at[pl.ds(base + 16, 16)], ws1)
        w0.wait()
        g2 = gstart(32, 8, a8, gs0)
        g2.wait()
        w2 = pltpu.async_copy(a8, out_hbm.at[pl.ds(base + 32, 8)], ws0)
        w1.wait()
        w2.wait()

    return _sc_gather


def kernel(x_embed, prompt, prompt_key):
    x_t = jnp.transpose(x_embed, (1, 0, 2))
    table = prompt.reshape(N_LAYERS * 2 * POOL, ROW)
    sim, idx, bkn, rowid, rsim = _routing_call(x_t, prompt_key)
    flat = _sc_gather_fn()(table, rowid.reshape(N_ROWS))
    bp = flat.reshape(N_LAYERS, 2, BATCH, TOP_K, LENGTH, N_HEADS,
                      EMBED_DIM // N_HEADS)
    bp = bp.reshape(N_LAYERS, BATCH, 2, TOP_K * LENGTH, N_HEADS,
                    EMBED_DIM // N_HEADS)
    return (bp, sim, idx, bkn, rsim[0, 0])
